# baseline (speedup 1.0000x reference)
import jax
import jax.numpy as jnp
from jax.experimental import pallas as pl

_DU = 20
_DZ = 5
_N = _DU * _DZ


def _body(x_ref, w0, b0, w1, b1, w2, b2, w3, b3,
          f1_ref, g1_ref, f2_ref, g2_ref):
    bb = x_ref.shape[0]
    x = x_ref[...]

    pieces = []
    for j in range(_DU):
        jm, jp = (j - 1) % _DU, (j + 1) % _DU
        h = (x[:, jm:jm + 1] * w0[0:1, :]
             + x[:, j:j + 1] * w0[1:2, :]
             + x[:, jp:jp + 1] * w0[2:3, :]) + b0[...]
        pieces.append(h)
    h = jnp.maximum(jnp.concatenate(pieces, axis=0), 0.0)

    h = jnp.maximum(
        jnp.dot(h, w1[...], preferred_element_type=jnp.float32) + b1[...], 0.0)
    h = jnp.maximum(
        jnp.dot(h, w2[...], preferred_element_type=jnp.float32) + b2[...], 0.0)
    out = jnp.dot(h, w3[...], preferred_element_type=jnp.float32) + b3[...]

    g1_ref[...] = jnp.zeros_like(g1_ref)
    g2_ref[...] = jnp.zeros_like(g2_ref)

    for j in range(_DU):
        s = out[j * bb:(j + 1) * bb, :]

        f1_ref[:, j:j + 1] = s[:, 0:1]
        f2_ref[:, _DZ * j:_DZ * (j + 1)] = s[:, 16:16 + _DZ]

        off = (_DZ * (j - 1)) % _N
        w15 = min(3 * _DZ, _N - off)
        g1_ref[:, j:j + 1, off:off + w15] = s[:, None, 1:1 + w15]
        if w15 < 3 * _DZ:
            g1_ref[:, j:j + 1, 0:3 * _DZ - w15] = s[:, None, 1 + w15:16]

        off = (_DZ * (j - 2)) % _N
        w25 = min(5 * _DZ, _N - off)
        for z in range(_DZ):
            c0 = 16 + _DZ + 25 * z
            r = _DZ * j + z
            g2_ref[:, r:r + 1, off:off + w25] = s[:, None, c0:c0 + w25]
            if w25 < 5 * _DZ:
                g2_ref[:, r:r + 1, 0:5 * _DZ - w25] = s[:, None, c0 + w25:c0 + 25]


def kernel(x, w1_0, b1_0, w1_1, b1_1, w1_2, b1_2, w1_3, b1_3,
           w2_0, b2_0, w2_1, b2_1, w2_2, b2_2, w2_3, b2_3):
    batch = x.shape[0]
    bb = 256 if batch % 256 == 0 else batch
    grid = (batch // bb,)
    f32 = jnp.float32

    w0 = jnp.concatenate([w1_0.T, w2_0.T], axis=1)
    b0 = jnp.concatenate([b1_0, b2_0]).reshape(1, -1)
    w1 = jnp.zeros((32, 64), f32).at[:16, :32].set(w1_1.T).at[16:, 32:].set(w2_1.T)
    b1 = jnp.concatenate([b1_1, b2_1]).reshape(1, -1)
    w2 = jnp.zeros((64, 32), f32).at[:32, :16].set(w1_2.T).at[32:, 16:].set(w2_2.T)
    b2 = jnp.concatenate([b1_2, b2_2]).reshape(1, -1)
    w3 = jnp.zeros((32, 146), f32).at[:16, :16].set(w1_3.T).at[16:, 16:].set(w2_3.T)
    b3 = jnp.concatenate([b1_3, b2_3]).reshape(1, -1)
    ws = [w0, b0, w1, b1, w2, b2, w3, b3]

    def wspec(a):
        return pl.BlockSpec(a.shape, lambda i: (0,) * a.ndim)

    f1, g1, f2, g2 = pl.pallas_call(
        _body,
        grid=grid,
        in_specs=[pl.BlockSpec((bb, _DU), lambda i: (i, 0))]
                  + [wspec(a) for a in ws],
        out_specs=[
            pl.BlockSpec((bb, _DU), lambda i: (i, 0)),
            pl.BlockSpec((bb, _DU, _N), lambda i: (i, 0, 0)),
            pl.BlockSpec((bb, _N), lambda i: (i, 0)),
            pl.BlockSpec((bb, _N, _N), lambda i: (i, 0, 0)),
        ],
        out_shape=[
            jax.ShapeDtypeStruct((batch, _DU), x.dtype),
            jax.ShapeDtypeStruct((batch, _DU, _N), x.dtype),
            jax.ShapeDtypeStruct((batch, _N), x.dtype),
            jax.ShapeDtypeStruct((batch, _N, _N), x.dtype),
        ],
    )(x, *ws)

    return (f1.reshape(batch, _DU, 1), g1, f2.reshape(batch, _N, 1), g2)


# speedup vs baseline: 1.7843x; 1.7843x over previous
import jax
import jax.numpy as jnp
from jax.experimental import pallas as pl

_DU = 20
_DZ = 5
_N = _DU * _DZ


def _body(x_ref, w0, b0, w1, b1, w2, b2, w3, b3,
          f1_ref, g1_ref, f2_ref, g2_ref):
    bb = x_ref.shape[0]
    x = x_ref[...]
    xm = jnp.concatenate([x[:, -1:], x[:, :-1]], axis=1)
    xp = jnp.concatenate([x[:, 1:], x[:, :1]], axis=1)
    xl = jnp.stack([xm, x, xp], axis=-1).reshape(bb * _DU, 3)

    h = jnp.maximum(
        jnp.dot(xl, w0[...], preferred_element_type=jnp.float32) + b0[...], 0.0)
    h = jnp.maximum(
        jnp.dot(h, w1[...], preferred_element_type=jnp.float32) + b1[...], 0.0)
    h = jnp.maximum(
        jnp.dot(h, w2[...], preferred_element_type=jnp.float32) + b2[...], 0.0)
    out = jnp.dot(h, w3[...], preferred_element_type=jnp.float32) + b3[...]

    out3 = out.reshape(bb, _DU, 146)
    f1_ref[...] = out3[:, :, 0]
    f2_ref[...] = out3[:, :, 16:16 + _DZ].reshape(bb, _N)
    vals1 = out3[:, :, 1:16]
    vals2 = out3[:, :, 16 + _DZ:].reshape(bb, _DU, _DZ, 5 * _DZ)

    g1_ref[...] = jnp.zeros_like(g1_ref)
    g2_ref[...] = jnp.zeros_like(g2_ref)

    for j in range(_DU):
        off = (_DZ * (j - 1)) % _N
        w15 = min(3 * _DZ, _N - off)
        g1_ref[:, j:j + 1, off:off + w15] = vals1[:, j:j + 1, :w15]
        if w15 < 3 * _DZ:
            g1_ref[:, j:j + 1, 0:3 * _DZ - w15] = vals1[:, j:j + 1, w15:]

        off = (_DZ * (j - 2)) % _N
        w25 = min(5 * _DZ, _N - off)
        v = vals2[:, j]
        g2_ref[:, _DZ * j:_DZ * (j + 1), off:off + w25] = v[:, :, :w25]
        if w25 < 5 * _DZ:
            g2_ref[:, _DZ * j:_DZ * (j + 1), 0:5 * _DZ - w25] = v[:, :, w25:]


def kernel(x, w1_0, b1_0, w1_1, b1_1, w1_2, b1_2, w1_3, b1_3,
           w2_0, b2_0, w2_1, b2_1, w2_2, b2_2, w2_3, b2_3):
    batch = x.shape[0]
    bb = 128 if batch % 128 == 0 else batch
    grid = (batch // bb,)
    f32 = jnp.float32

    w0 = jnp.concatenate([w1_0.T, w2_0.T], axis=1)
    b0 = jnp.concatenate([b1_0, b2_0]).reshape(1, -1)
    w1 = jnp.zeros((32, 64), f32).at[:16, :32].set(w1_1.T).at[16:, 32:].set(w2_1.T)
    b1 = jnp.concatenate([b1_1, b2_1]).reshape(1, -1)
    w2 = jnp.zeros((64, 32), f32).at[:32, :16].set(w1_2.T).at[32:, 16:].set(w2_2.T)
    b2 = jnp.concatenate([b1_2, b2_2]).reshape(1, -1)
    w3 = jnp.zeros((32, 146), f32).at[:16, :16].set(w1_3.T).at[16:, 16:].set(w2_3.T)
    b3 = jnp.concatenate([b1_3, b2_3]).reshape(1, -1)
    ws = [w0, b0, w1, b1, w2, b2, w3, b3]

    def wspec(a):
        return pl.BlockSpec(a.shape, lambda i: (0,) * a.ndim)

    f1, g1, f2, g2 = pl.pallas_call(
        _body,
        grid=grid,
        in_specs=[pl.BlockSpec((bb, _DU), lambda i: (i, 0))]
                  + [wspec(a) for a in ws],
        out_specs=[
            pl.BlockSpec((bb, _DU), lambda i: (i, 0)),
            pl.BlockSpec((bb, _DU, _N), lambda i: (i, 0, 0)),
            pl.BlockSpec((bb, _N), lambda i: (i, 0)),
            pl.BlockSpec((bb, _N, _N), lambda i: (i, 0, 0)),
        ],
        out_shape=[
            jax.ShapeDtypeStruct((batch, _DU), x.dtype),
            jax.ShapeDtypeStruct((batch, _DU, _N), x.dtype),
            jax.ShapeDtypeStruct((batch, _N), x.dtype),
            jax.ShapeDtypeStruct((batch, _N, _N), x.dtype),
        ],
    )(x, *ws)

    return (f1.reshape(batch, _DU, 1), g1, f2.reshape(batch, _N, 1), g2)
